# baseline (speedup 1.0000x reference)
import functools

import jax
import jax.numpy as jnp
from jax import lax
from jax.experimental import pallas as pl
from jax.experimental.pallas import tpu as pltpu
from jax.experimental.pallas import tpu_sc as plsc

NUM_TYPES = 1000
NC = 2
NS = 16
L = 16
NW = NC * NS
WSZ = NUM_TYPES * NUM_TYPES
NBUF = 4


def _sc_gather_sum(seq, pad16, w_flat, per_tile, chunk):
    nchunk = per_tile // chunk
    groups = chunk // L
    mesh = plsc.VectorSubcoreMesh(core_axis_name="c", subcore_axis_name="s")

    @functools.partial(
        pl.kernel,
        out_type=jax.ShapeDtypeStruct((NW, L), jnp.float32),
        mesh=mesh,
        scratch_types=[
            pltpu.VMEM((chunk + L,), jnp.int32),
            pltpu.VMEM((chunk + L,), jnp.int32),
            pltpu.VMEM((L,), jnp.int32),
            pltpu.VMEM((L,), jnp.float32),
            pltpu.VMEM((chunk,), jnp.int32),
            pltpu.VMEM((chunk,), jnp.int32),
            pltpu.VMEM((chunk,), jnp.int32),
            pltpu.VMEM((chunk,), jnp.int32),
            pltpu.VMEM((chunk,), jnp.float32),
            pltpu.VMEM((chunk,), jnp.float32),
            pltpu.VMEM((chunk,), jnp.float32),
            pltpu.VMEM((chunk,), jnp.float32),
            pltpu.VMEM((L,), jnp.float32),
            pltpu.VMEM_SHARED((WSZ,), jnp.float32),
            pltpu.SemaphoreType.DMA,
            pltpu.SemaphoreType.DMA,
            pltpu.SemaphoreType.DMA,
            pltpu.SemaphoreType.DMA,
            pltpu.SemaphoreType.DMA,
            pltpu.SemaphoreType.DMA,
            pltpu.SemaphoreType.DMA,
            pltpu.SemaphoreType.DMA,
        ],
    )
    def k(seq_hbm, pad_hbm, w_hbm, out_hbm,
          seqa, seqb, pad_v, w0_v, idx0, idx1, idx2, idx3,
          val0, val1, val2, val3, acc_v, w_sh,
          sem0, sem1, sem2, sem0b, sem1b, sem2b, sseq, swst):
        sid = lax.axis_index("s")
        wid = sid * NC + lax.axis_index("c")
        base = wid * per_tile
        seq_bufs = (seqa, seqb)
        idx_bufs = (idx0, idx1, idx2, idx3)
        val_bufs = (val0, val1, val2, val3)
        sems = (sem0, sem1, sem2)
        semsb = (sem0b, sem1b, sem2b)

        @pl.when(sid == 0)
        def _():
            pltpu.async_copy(w_hbm, w_sh, swst)

        def start_seq(ci):
            buf = seq_bufs[ci % 2]
            off = base + ci * chunk
            if ci == nchunk - 1:
                @pl.when(wid == NW - 1)
                def _():
                    pltpu.async_copy(seq_hbm.at[pl.ds(off, chunk)],
                                     buf.at[pl.ds(0, chunk)], sseq)
                    pltpu.sync_copy(pad_hbm, buf.at[pl.ds(chunk, L)])

                @pl.when(wid != NW - 1)
                def _():
                    pltpu.async_copy(seq_hbm.at[pl.ds(off, chunk + L)],
                                     buf, sseq)
            else:
                pltpu.async_copy(seq_hbm.at[pl.ds(off, chunk + L)], buf, sseq)

        def wait_seq(ci):
            buf = seq_bufs[ci % 2]
            if ci == nchunk - 1:
                @pl.when(wid == NW - 1)
                def _():
                    pltpu.make_async_copy(seq_hbm.at[pl.ds(0, chunk)],
                                          buf.at[pl.ds(0, chunk)], sseq).wait()

                @pl.when(wid != NW - 1)
                def _():
                    pltpu.make_async_copy(seq_hbm.at[pl.ds(0, chunk + L)],
                                          buf, sseq).wait()
            else:
                pltpu.make_async_copy(seq_hbm.at[pl.ds(0, chunk + L)],
                                      buf, sseq).wait()

        def build(ci, cnt):
            seq_v = seq_bufs[ci % 2]
            idx_ref = idx_bufs[ci % NBUF]

            lanes = lax.iota(jnp.int32, L)

            def body(j, c):
                a = seq_v[pl.ds(j * L, L)]
                b = seq_v[pl.ds(j * L + 1, L)]
                m = (a == pad) | (b == pad)
                f = jnp.where(m, lanes, a * NUM_TYPES + b)
                idx_ref[pl.ds(j * L, L)] = f
                return c + jnp.where(m, 1, 0)

            return lax.fori_loop(0, groups, body, cnt, unroll=4)

        def accum(ci, acc):
            val_ref = val_bufs[ci % NBUF]

            def body(j, a):
                return a + val_ref[pl.ds(j * L, L)]

            return lax.fori_loop(0, groups, body, acc, unroll=8)

        half = chunk // 2

        def start_gather(ci):
            idx_ref = idx_bufs[ci % NBUF]
            val_ref = val_bufs[ci % NBUF]
            h1 = pltpu.async_copy(
                w_sh.at[idx_ref.at[pl.ds(0, half)]],
                val_ref.at[pl.ds(0, half)], sems[ci % 3])
            h2 = pltpu.async_copy(
                w_sh.at[idx_ref.at[pl.ds(half, half)]],
                val_ref.at[pl.ds(half, half)], semsb[ci % 3])
            return (h1, h2)

        cnt = jnp.zeros((L,), jnp.int32)
        acc = jnp.zeros((L,), jnp.float32)
        start_seq(0)
        pltpu.sync_copy(pad_hbm, pad_v)
        pltpu.sync_copy(w_hbm.at[pl.ds(0, L)], w0_v)
        pad = pad_v[...]
        pending = [None, None, None]
        for ci in range(nchunk):
            wait_seq(ci)
            if ci + 1 < nchunk:
                start_seq(ci + 1)
            cnt = build(ci, cnt)
            if ci == 0:
                continue
            if ci == 1:
                @pl.when(sid == 0)
                def _():
                    pltpu.make_async_copy(w_hbm, w_sh, swst).wait()

                plsc.subcore_barrier()
                pending[0] = start_gather(0)
                pending[1] = start_gather(1)
                continue
            if ci >= 3:
                for h in pending[(ci - 3) % 3]:
                    h.wait()
            new = start_gather(ci)
            if ci >= 3:
                acc = accum(ci - 3, acc)
            pending[ci % 3] = new
        for ci in range(nchunk - 3, nchunk):
            for h in pending[ci % 3]:
                h.wait()
            acc = accum(ci, acc)
        acc_v[...] = acc - cnt.astype(jnp.float32) * w0_v[...]
        pltpu.sync_copy(acc_v, out_hbm.at[wid])

    return k(seq, pad16, w_flat)


def kernel(sequence, padding_idx, W):
    n = sequence.shape[0]
    per_tile = n // NW
    chunk = per_tile
    for c in (6400, 3200, 1600, 800, 400, 200, 100):
        if per_tile % c == 0 and 10 * c + 8 * L <= 64200:
            chunk = c
            break
    pad16 = jnp.full((L,), padding_idx, dtype=sequence.dtype)
    w_flat = W.reshape(-1)
    parts = _sc_gather_sum(sequence, pad16, w_flat, per_tile, chunk)
    return -jnp.sum(parts)


# speedup vs baseline: 1.0007x; 1.0007x over previous
import functools

import jax
import jax.numpy as jnp
from jax import lax
from jax.experimental import pallas as pl
from jax.experimental.pallas import tpu as pltpu
from jax.experimental.pallas import tpu_sc as plsc

NUM_TYPES = 1000
NC = 2
NS = 16
L = 16
NW = NC * NS
WSZ = NUM_TYPES * NUM_TYPES
NBUF = 4


def _sc_gather_sum(seq, pad16, w_flat, per_tile, chunk):
    nchunk = per_tile // chunk
    groups = chunk // L
    mesh = plsc.VectorSubcoreMesh(core_axis_name="c", subcore_axis_name="s")

    @functools.partial(
        pl.kernel,
        out_type=jax.ShapeDtypeStruct((NW, L), jnp.float32),
        mesh=mesh,
        scratch_types=[
            pltpu.VMEM((chunk + L,), jnp.int32),
            pltpu.VMEM((chunk + L,), jnp.int32),
            pltpu.VMEM((L,), jnp.int32),
            pltpu.VMEM((L,), jnp.float32),
            pltpu.VMEM((chunk,), jnp.int32),
            pltpu.VMEM((chunk,), jnp.int32),
            pltpu.VMEM((chunk,), jnp.int32),
            pltpu.VMEM((chunk,), jnp.int32),
            pltpu.VMEM((chunk,), jnp.float32),
            pltpu.VMEM((chunk,), jnp.float32),
            pltpu.VMEM((chunk,), jnp.float32),
            pltpu.VMEM((chunk,), jnp.float32),
            pltpu.VMEM((L,), jnp.float32),
            pltpu.VMEM_SHARED((WSZ,), jnp.float32),
            pltpu.SemaphoreType.DMA,
            pltpu.SemaphoreType.DMA,
            pltpu.SemaphoreType.DMA,
            pltpu.SemaphoreType.DMA,
            pltpu.SemaphoreType.DMA,
        ],
    )
    def k(seq_hbm, pad_hbm, w_hbm, out_hbm,
          seqa, seqb, pad_v, w0_v, idx0, idx1, idx2, idx3,
          val0, val1, val2, val3, acc_v, w_sh,
          sem0, sem1, sem2, sseq, swst):
        sid = lax.axis_index("s")
        wid = sid * NC + lax.axis_index("c")
        base = wid * per_tile
        seq_bufs = (seqa, seqb)
        idx_bufs = (idx0, idx1, idx2, idx3)
        val_bufs = (val0, val1, val2, val3)
        sems = (sem0, sem1, sem2)

        @pl.when(sid == 0)
        def _():
            pltpu.async_copy(w_hbm, w_sh, swst)

        def start_seq(ci):
            buf = seq_bufs[ci % 2]
            off = base + ci * chunk
            if ci == nchunk - 1:
                @pl.when(wid == NW - 1)
                def _():
                    pltpu.async_copy(seq_hbm.at[pl.ds(off, chunk)],
                                     buf.at[pl.ds(0, chunk)], sseq)
                    pltpu.sync_copy(pad_hbm, buf.at[pl.ds(chunk, L)])

                @pl.when(wid != NW - 1)
                def _():
                    pltpu.async_copy(seq_hbm.at[pl.ds(off, chunk + L)],
                                     buf, sseq)
            else:
                pltpu.async_copy(seq_hbm.at[pl.ds(off, chunk + L)], buf, sseq)

        def wait_seq(ci):
            buf = seq_bufs[ci % 2]
            if ci == nchunk - 1:
                @pl.when(wid == NW - 1)
                def _():
                    pltpu.make_async_copy(seq_hbm.at[pl.ds(0, chunk)],
                                          buf.at[pl.ds(0, chunk)], sseq).wait()

                @pl.when(wid != NW - 1)
                def _():
                    pltpu.make_async_copy(seq_hbm.at[pl.ds(0, chunk + L)],
                                          buf, sseq).wait()
            else:
                pltpu.make_async_copy(seq_hbm.at[pl.ds(0, chunk + L)],
                                      buf, sseq).wait()

        def build(ci, cnt):
            seq_v = seq_bufs[ci % 2]
            idx_ref = idx_bufs[ci % NBUF]

            lanes = lax.iota(jnp.int32, L)

            def body(j, c):
                a = seq_v[pl.ds(j * L, L)]
                b = seq_v[pl.ds(j * L + 1, L)]
                m = (a == pad) | (b == pad)
                f = jnp.where(m, lanes, a * NUM_TYPES + b)
                idx_ref[pl.ds(j * L, L)] = f
                return c + jnp.where(m, 1, 0)

            return lax.fori_loop(0, groups, body, cnt, unroll=4)

        def accum(ci, acc):
            val_ref = val_bufs[ci % NBUF]

            def body(j, a):
                return a + val_ref[pl.ds(j * L, L)]

            return lax.fori_loop(0, groups, body, acc, unroll=8)

        def start_gather(ci):
            return pltpu.async_copy(
                w_sh.at[idx_bufs[ci % NBUF]], val_bufs[ci % NBUF],
                sems[ci % 3])

        cnt = jnp.zeros((L,), jnp.int32)
        acc = jnp.zeros((L,), jnp.float32)
        start_seq(0)
        pltpu.sync_copy(pad_hbm, pad_v)
        pltpu.sync_copy(w_hbm.at[pl.ds(0, L)], w0_v)
        pad = pad_v[...]
        pending = [None, None, None]
        for ci in range(nchunk):
            wait_seq(ci)
            if ci + 1 < nchunk:
                start_seq(ci + 1)
            cnt = build(ci, cnt)
            if ci == 0:
                continue
            if ci == 1:
                @pl.when(sid == 0)
                def _():
                    pltpu.make_async_copy(w_hbm, w_sh, swst).wait()

                plsc.subcore_barrier()
                pending[0] = start_gather(0)
                pending[1] = start_gather(1)
                continue
            if ci >= 3:
                pending[(ci - 3) % 3].wait()
            new = start_gather(ci)
            if ci >= 3:
                acc = accum(ci - 3, acc)
            pending[ci % 3] = new
        for ci in range(nchunk - 3, nchunk):
            pending[ci % 3].wait()
            acc = accum(ci, acc)
        acc_v[...] = acc - cnt.astype(jnp.float32) * w0_v[...]
        pltpu.sync_copy(acc_v, out_hbm.at[wid])

    return k(seq, pad16, w_flat)


def kernel(sequence, padding_idx, W):
    n = sequence.shape[0]
    per_tile = n // NW
    chunk = per_tile
    for c in (6400, 3200, 1600, 800, 400, 200, 100):
        if per_tile % c == 0 and 10 * c + 8 * L <= 64200:
            chunk = c
            break
    pad16 = jnp.full((L,), padding_idx, dtype=sequence.dtype)
    w_flat = W.reshape(-1)
    parts = _sc_gather_sum(sequence, pad16, w_flat, per_tile, chunk)
    return -jnp.sum(parts)
